# baseline (speedup 1.0000x reference)
import jax
import jax.numpy as jnp
from jax import lax
from jax.experimental import pallas as pl
from jax.experimental.pallas import tpu as pltpu
from jax.experimental.pallas import tpu_sc as plsc

N = 10000
E = 320000
D = 128
ED = 4
CH = 128
NC = 2
NS = 16
NW = NC * NS

TE = 3200
TEB = TE // CH
NP = 10240
TN = 2048

S = 4
SE = E // S
_CHUNKS = SE // CH
_ITERS = -(-_CHUNKS // NW)
_CHP = _ITERS * NW
SEP = _CHP * CH
_ZR = 200
_ZCH = N // _ZR
_ZIT = -(-_ZCH // NS)


def _rsqrt(x):
    i = plsc.bitcast(x, jnp.int32)
    i = jnp.full((16,), 0x5F3759DF, jnp.int32) - (i >> 1)
    y = plsc.bitcast(i, jnp.float32)
    half = x * 0.5
    for _ in range(3):
        y = y * (1.5 - half * y * y)
    return y


def _gather_body(htbl, post, idx3, gr_hbm, gc_hbm, rad_hbm,
                 pos_v, idx_v0, idx_v1, rr0, rr1, rc0, rc1, rad_v0, rad_v1,
                 semg0, semg1, semo0, semo1):
    c = lax.axis_index("c")
    s = lax.axis_index("s")

    pltpu.sync_copy(post, pos_v)
    idx_v = (idx_v0, idx_v1)
    rr = (rr0, rr1)
    rc = (rc0, rc1)
    rad_v = (rad_v0, rad_v1)
    semg = (semg0, semg1)
    semo = (semo0, semo1)

    @pl.loop(0, _ITERS, step=2)
    def _(t):
        for b in range(2):
            tt = t + b
            k = ((tt * NS + s) * NC + c).astype(jnp.int32)

            @pl.when(tt >= 2)
            def _():
                pltpu.make_async_copy(gr_hbm.at[pl.ds(0, CH)], rr[b],
                                      semo[b]).wait()
                pltpu.make_async_copy(gc_hbm.at[pl.ds(0, CH)], rc[b],
                                      semo[b]).wait()
                pltpu.make_async_copy(rad_hbm.at[:, pl.ds(0, CH)], rad_v[b],
                                      semo[b]).wait()

            pltpu.sync_copy(idx3.at[k], idx_v[b])
            cpr = pltpu.async_copy(htbl.at[idx_v[b].at[0]], rr[b], semg[b])
            cpc = pltpu.async_copy(htbl.at[idx_v[b].at[1]], rc[b], semg[b])
            for j in range(CH // 16):
                ir = idx_v[b][0, pl.ds(j * 16, 16)]
                ic = idx_v[b][1, pl.ds(j * 16, 16)]
                rx = (plsc.load_gather(pos_v, [ir])
                      - plsc.load_gather(pos_v, [ic]))
                ry = (plsc.load_gather(pos_v, [ir + N])
                      - plsc.load_gather(pos_v, [ic + N]))
                rz = (plsc.load_gather(pos_v, [ir + 2 * N])
                      - plsc.load_gather(pos_v, [ic + 2 * N]))
                rn2 = rx * rx + ry * ry + rz * rz
                inv = _rsqrt(rn2)
                rn = jnp.maximum(rn2 * inv, 1e-8)
                rad_v[b][0, pl.ds(j * 16, 16)] = rn
                rad_v[b][1, pl.ds(j * 16, 16)] = rx * inv
                rad_v[b][2, pl.ds(j * 16, 16)] = ry * inv
                rad_v[b][3, pl.ds(j * 16, 16)] = rz * inv
            cpr.wait()
            cpc.wait()
            pltpu.async_copy(rr[b], gr_hbm.at[pl.ds(k * CH, CH)], semo[b])
            pltpu.async_copy(rc[b], gc_hbm.at[pl.ds(k * CH, CH)], semo[b])
            pltpu.async_copy(rad_v[b], rad_hbm.at[:, pl.ds(k * CH, CH)],
                             semo[b])

    for b in range(2):
        pltpu.make_async_copy(gr_hbm.at[pl.ds(0, CH)], rr[b], semo[b]).wait()
        pltpu.make_async_copy(gc_hbm.at[pl.ds(0, CH)], rc[b], semo[b]).wait()
        pltpu.make_async_copy(rad_hbm.at[:, pl.ds(0, CH)], rad_v[b],
                              semo[b]).wait()


def _sc_gather(htbl, post, idx3):
    mesh = plsc.VectorSubcoreMesh(core_axis_name="c", subcore_axis_name="s")
    f = pl.kernel(
        _gather_body,
        out_type=(
            jax.ShapeDtypeStruct((SEP, D), jnp.float32),
            jax.ShapeDtypeStruct((SEP, D), jnp.float32),
            jax.ShapeDtypeStruct((4, SEP), jnp.float32),
        ),
        mesh=mesh,
        compiler_params=pltpu.CompilerParams(needs_layout_passes=False),
        scratch_types=[
            pltpu.VMEM((3 * N,), jnp.float32),
            pltpu.VMEM((2, CH), jnp.int32),
            pltpu.VMEM((2, CH), jnp.int32),
            pltpu.VMEM((CH, D), jnp.float32),
            pltpu.VMEM((CH, D), jnp.float32),
            pltpu.VMEM((CH, D), jnp.float32),
            pltpu.VMEM((CH, D), jnp.float32),
            pltpu.VMEM((4, CH), jnp.float32),
            pltpu.VMEM((4, CH), jnp.float32),
            pltpu.SemaphoreType.DMA,
            pltpu.SemaphoreType.DMA,
            pltpu.SemaphoreType.DMA,
            pltpu.SemaphoreType.DMA,
        ],
    )
    return f(htbl, post, idx3)


def _scatter_body(m_hbm, cu_hbm, ridx2, z_hbm, agg_hbm, pout_hbm,
                  idx_u0, idx_u1, rows_u0, rows_u1, cu_u0, cu_u1, pacc_v,
                  seml0, seml1, acc):
    c = lax.axis_index("c")
    s = lax.axis_index("s")

    @pl.loop(0, _ZIT)
    def _(i):
        j = s + NS * i

        @pl.when(j < _ZCH)
        def _():
            pltpu.sync_copy(z_hbm, acc.at[pl.ds(j * _ZR, _ZR)])

    @pl.loop(0, 3 * N // 16)
    def _(i):
        pacc_v[pl.ds(i * 16, 16)] = jnp.zeros((16,), jnp.float32)

    plsc.subcore_barrier()

    @pl.loop(0, _ITERS)
    def _(t):
        k = ((t * NS + s) * NC + c).astype(jnp.int32)

        @pl.when(k < _CHUNKS)
        def _():
            pltpu.sync_copy(ridx2.at[k], idx_u0)
            pltpu.sync_copy(m_hbm.at[pl.ds(k * CH, CH)], rows_u0)
            pltpu.sync_copy(cu_hbm.at[:, pl.ds(k * CH, CH)], cu_u0)
            pltpu.sync_copy(rows_u0, acc.at[idx_u0], add=True)
            for j in range(CH // 16):
                ir = idx_u0[pl.ds(j * 16, 16)]
                plsc.addupdate_scatter(pacc_v, [ir],
                                       cu_u0[0, pl.ds(j * 16, 16)])
                plsc.addupdate_scatter(pacc_v, [ir + N],
                                       cu_u0[1, pl.ds(j * 16, 16)])
                plsc.addupdate_scatter(pacc_v, [ir + 2 * N],
                                       cu_u0[2, pl.ds(j * 16, 16)])

    plsc.subcore_barrier()

    @pl.loop(0, _ZIT)
    def _(i):
        j = s + NS * i

        @pl.when(j < _ZCH)
        def _():
            pltpu.sync_copy(acc.at[pl.ds(j * _ZR, _ZR)],
                            agg_hbm.at[pl.ds(c * NP + j * _ZR, _ZR)])

    w = s * NC + c
    pltpu.sync_copy(pacc_v, pout_hbm.at[w])


def _sc_scatter(m, cu, ridx2, zrows):
    mesh = plsc.VectorSubcoreMesh(core_axis_name="c", subcore_axis_name="s")
    f = pl.kernel(
        _scatter_body,
        out_type=(
            jax.ShapeDtypeStruct((2 * NP, D), jnp.float32),
            jax.ShapeDtypeStruct((NW, 3 * N), jnp.float32),
        ),
        mesh=mesh,
        compiler_params=pltpu.CompilerParams(needs_layout_passes=False),
        scratch_types=[
            pltpu.VMEM((CH,), jnp.int32),
            pltpu.VMEM((CH,), jnp.int32),
            pltpu.VMEM((CH, D), jnp.float32),
            pltpu.VMEM((CH, D), jnp.float32),
            pltpu.VMEM((4, CH), jnp.float32),
            pltpu.VMEM((4, CH), jnp.float32),
            pltpu.VMEM((3 * N,), jnp.float32),
            pltpu.SemaphoreType.DMA,
            pltpu.SemaphoreType.DMA,
            pltpu.VMEM_SHARED((N, D), jnp.float32),
        ],
    )
    return f(m, cu, ridx2, zrows)


def _silu(x):
    return x * jax.nn.sigmoid(x)


def _edge_body(gr, gc, rad, ea, w1a, w1b, w1n, w1e, b1, w2, b2, w5, b5, w6t,
               b6, m_out, cu_out):
    radt = jnp.transpose(rad[...], (1, 0))
    rn = radt[:, 0:1]
    rnorm = radt[:, 1:4]
    eat = jnp.transpose(ea[...], (1, 0))

    x = (jnp.dot(gr[...], w1a[...], preferred_element_type=jnp.float32)
         + jnp.dot(gc[...], w1b[...], preferred_element_type=jnp.float32)
         + rn * w1n[...]
         + jnp.dot(eat, w1e[...], preferred_element_type=jnp.float32)
         + b1[...])
    x = _silu(x)
    m = _silu(jnp.dot(x, w2[...], preferred_element_type=jnp.float32) + b2[...])
    y = _silu(jnp.dot(m, w5[...], preferred_element_type=jnp.float32) + b5[...])
    cd = jnp.sum(y * w6t[...], axis=1, keepdims=True) + b6[...]
    m_out[...] = m
    cu4 = jnp.concatenate(
        [cd * rnorm, jnp.zeros((TE, 1), jnp.float32)], axis=1)
    cu_out[...] = jnp.transpose(cu4, (1, 0))


def _tc_edge(gr, gc, rad, ea, w1a, w1b, w1n, w1e, b1, w2, b2, w5, b5, w6t,
             b6):
    grid = (SE // TE,)
    full = lambda shape: pl.BlockSpec(shape, lambda i: tuple(0 for _ in shape))
    return pl.pallas_call(
        _edge_body,
        grid=grid,
        in_specs=[
            pl.BlockSpec((TE, D), lambda i: (i, 0)),
            pl.BlockSpec((TE, D), lambda i: (i, 0)),
            pl.BlockSpec((4, TE), lambda i: (0, i)),
            pl.BlockSpec((ED, TE), lambda i: (0, i)),
            full((D, D)), full((D, D)), full((1, D)), full((ED, D)),
            full((1, D)), full((D, D)), full((1, D)), full((D, D)),
            full((1, D)), full((1, D)), full((1, 1)),
        ],
        out_specs=[
            pl.BlockSpec((TE, D), lambda i: (i, 0)),
            pl.BlockSpec((4, TE), lambda i: (0, i)),
        ],
        out_shape=[
            jax.ShapeDtypeStruct((SEP, D), jnp.float32),
            jax.ShapeDtypeStruct((4, SEP), jnp.float32),
        ],
        compiler_params=pltpu.CompilerParams(
            dimension_semantics=("arbitrary",)),
    )(gr, gc, rad, ea, w1a, w1b, w1n, w1e, b1, w2, b2, w5, b5, w6t, b6)


def _node_body(h, *rest):
    parts = rest[:2 * S]
    w3a, w3b, b3, w4, b4 = rest[2 * S:2 * S + 5]
    hn = rest[2 * S + 5]
    agg = parts[0][...]
    for pr in parts[1:]:
        agg = agg + pr[...]
    x = _silu(jnp.dot(h[...], w3a[...], preferred_element_type=jnp.float32)
              + jnp.dot(agg, w3b[...], preferred_element_type=jnp.float32)
              + b3[...])
    hn[...] = h[...] + jnp.dot(x, w4[...],
                               preferred_element_type=jnp.float32) + b4[...]


def _tc_node(h, parts_list, w3a, w3b, b3, w4, b4):
    grid = (NP // TN,)
    full = lambda shape: pl.BlockSpec(shape, lambda i: tuple(0 for _ in shape))
    part_specs = []
    for _ in parts_list:
        part_specs.append(pl.BlockSpec((TN, D), lambda i: (i, 0)))
        part_specs.append(pl.BlockSpec((TN, D), lambda i: (i + NP // TN, 0)))
    part_args = []
    for pr in parts_list:
        part_args.extend([pr, pr])
    return pl.pallas_call(
        _node_body,
        grid=grid,
        in_specs=[pl.BlockSpec((TN, D), lambda i: (i, 0))] + part_specs + [
            full((D, D)), full((D, D)), full((1, D)), full((D, D)),
            full((1, D)),
        ],
        out_specs=pl.BlockSpec((TN, D), lambda i: (i, 0)),
        out_shape=jax.ShapeDtypeStruct((NP, D), jnp.float32),
        compiler_params=pltpu.CompilerParams(
            dimension_semantics=("arbitrary",)),
    )(h, *part_args, w3a, w3b, b3, w4, b4)


def kernel(h, pos, edge_attr, W1, b1, W2, b2, W3, b3, W4, b4, W5, b5, W6, b6,
           edge_index):
    post = pos.T.reshape(3 * N)
    pad_i = jnp.zeros((S, _CHP - _CHUNKS, CH), jnp.int32)
    idxr2 = jnp.concatenate(
        [edge_index[0].reshape(S, _CHUNKS, CH), pad_i], axis=1)
    idxc2 = jnp.concatenate(
        [edge_index[1].reshape(S, _CHUNKS, CH), pad_i], axis=1)
    idx3 = jnp.stack([idxr2, idxc2], axis=2)
    ridx2 = jnp.concatenate(
        [edge_index[0].reshape(S, _CHUNKS, CH),
         jnp.full((S, _CHP - _CHUNKS, CH), N, jnp.int32)], axis=1)

    w1a = W1[0:D]
    w1b = W1[D:2 * D]
    w1n = W1[2 * D:2 * D + 1]
    w1e = W1[2 * D + 1:]
    eat = edge_attr.T.reshape(ED, S, SE)
    zrows = jnp.zeros((_ZR, D), jnp.float32)

    parts_list = []
    pparts_list = []
    for si in range(S):
        gr, gc, rad = _sc_gather(h, post, idx3[si])
        m, cu = _tc_edge(gr, gc, rad, eat[:, si], w1a, w1b, w1n, w1e,
                         b1.reshape(1, D), W2, b2.reshape(1, D),
                         W5, b5.reshape(1, D),
                         W6.reshape(1, D), b6.reshape(1, 1))
        parts, pparts = _sc_scatter(m, cu, ridx2[si], zrows)
        parts_list.append(parts)
        pparts_list.append(pparts)

    hp = jnp.concatenate([h, jnp.zeros((NP - N, D), jnp.float32)], axis=0)
    h_new = _tc_node(hp, parts_list, W3[0:D], W3[D:2 * D],
                     b3.reshape(1, D), W4, b4.reshape(1, D))
    pd = pparts_list[0].sum(axis=0)
    for pp in pparts_list[1:]:
        pd = pd + pp.sum(axis=0)
    pos_new = pos + pd.reshape(3, N).T
    return (h_new[:N], pos_new)


# speedup vs baseline: 1.5599x; 1.5599x over previous
import jax
import jax.numpy as jnp
from jax import lax
from jax.experimental import pallas as pl
from jax.experimental.pallas import tpu as pltpu
from jax.experimental.pallas import tpu_sc as plsc

N = 10000
E = 320000
D = 128
ED = 4
CH = 128
NC = 2
NS = 16
NW = NC * NS

TE = 3200
NP = 10240
TN = 2048

S = 4
SE = E // S
_CHUNKS = SE // CH
_ITERS = -(-_CHUNKS // NW)
_ZR = 200
_ZCH = N // _ZR
_ZIT = -(-_ZCH // NS)


def _rsqrt(x):
    i = plsc.bitcast(x, jnp.int32)
    i = jnp.full((16,), 0x5F3759DF, jnp.int32) - (i >> 1)
    y = plsc.bitcast(i, jnp.float32)
    half = x * 0.5
    for _ in range(3):
        y = y * (1.5 - half * y * y)
    return y


def _gather_body(htbl, post, idxr2, idxc2, g_hbm, rad_hbm,
                 pos_v, idxr_v, idxc_v, rowsr_v, rowsc_v, rad_v, semr, semc):
    c = lax.axis_index("c")
    s = lax.axis_index("s")

    pltpu.sync_copy(post, pos_v)

    @pl.loop(0, _ITERS)
    def _(t):
        k = ((t * NS + s) * NC + c).astype(jnp.int32)

        @pl.when(k < _CHUNKS)
        def _():
            pltpu.sync_copy(idxr2.at[k], idxr_v)
            pltpu.sync_copy(idxc2.at[k], idxc_v)
            cpr = pltpu.async_copy(htbl.at[idxr_v], rowsr_v, semr)
            cpc = pltpu.async_copy(htbl.at[idxc_v], rowsc_v, semc)
            for j in range(CH // 16):
                ir = idxr_v[pl.ds(j * 16, 16)]
                ic = idxc_v[pl.ds(j * 16, 16)]
                rx = (plsc.load_gather(pos_v, [ir])
                      - plsc.load_gather(pos_v, [ic]))
                ry = (plsc.load_gather(pos_v, [ir + N])
                      - plsc.load_gather(pos_v, [ic + N]))
                rz = (plsc.load_gather(pos_v, [ir + 2 * N])
                      - plsc.load_gather(pos_v, [ic + 2 * N]))
                rn2 = rx * rx + ry * ry + rz * rz
                inv = _rsqrt(rn2)
                rn = jnp.maximum(rn2 * inv, 1e-8)
                rad_v[0, pl.ds(j * 16, 16)] = rn
                rad_v[1, pl.ds(j * 16, 16)] = rx * inv
                rad_v[2, pl.ds(j * 16, 16)] = ry * inv
                rad_v[3, pl.ds(j * 16, 16)] = rz * inv
            cpr.wait()
            cpc.wait()
            pltpu.sync_copy(rowsr_v, g_hbm.at[pl.ds(k * CH, CH)])
            pltpu.sync_copy(rowsc_v, g_hbm.at[pl.ds(SE + k * CH, CH)])
            pltpu.sync_copy(rad_v, rad_hbm.at[:, pl.ds(k * CH, CH)])


def _sc_gather(htbl, post, idxr2, idxc2):
    mesh = plsc.VectorSubcoreMesh(core_axis_name="c", subcore_axis_name="s")
    f = pl.kernel(
        _gather_body,
        out_type=(
            jax.ShapeDtypeStruct((2 * SE, D), jnp.float32),
            jax.ShapeDtypeStruct((4, SE), jnp.float32),
        ),
        mesh=mesh,
        compiler_params=pltpu.CompilerParams(needs_layout_passes=False),
        scratch_types=[
            pltpu.VMEM((3 * N,), jnp.float32),
            pltpu.VMEM((CH,), jnp.int32),
            pltpu.VMEM((CH,), jnp.int32),
            pltpu.VMEM((CH, D), jnp.float32),
            pltpu.VMEM((CH, D), jnp.float32),
            pltpu.VMEM((4, CH), jnp.float32),
            pltpu.SemaphoreType.DMA,
            pltpu.SemaphoreType.DMA,
        ],
    )
    return f(htbl, post, idxr2, idxc2)


def _scatter_body(m_hbm, cu_hbm, ridx2, z_hbm, agg_hbm, pout_hbm,
                  idx_v, rows_v, cu_v, pacc_v, sem, acc):
    c = lax.axis_index("c")
    s = lax.axis_index("s")

    @pl.loop(0, _ZIT)
    def _(i):
        j = s + NS * i

        @pl.when(j < _ZCH)
        def _():
            pltpu.sync_copy(z_hbm, acc.at[pl.ds(j * _ZR, _ZR)])

    @pl.loop(0, 3 * N // 16)
    def _(i):
        pacc_v[pl.ds(i * 16, 16)] = jnp.zeros((16,), jnp.float32)

    plsc.subcore_barrier()

    @pl.loop(0, _ITERS)
    def _(t):
        k = ((t * NS + s) * NC + c).astype(jnp.int32)

        @pl.when(k < _CHUNKS)
        def _():
            pltpu.sync_copy(ridx2.at[k], idx_v)
            pltpu.sync_copy(m_hbm.at[pl.ds(k * CH, CH)], rows_v)
            pltpu.sync_copy(cu_hbm.at[:, pl.ds(k * CH, CH)], cu_v)
            pltpu.sync_copy(rows_v, acc.at[idx_v], add=True)
            for j in range(CH // 16):
                ir = idx_v[pl.ds(j * 16, 16)]
                plsc.addupdate_scatter(pacc_v, [ir],
                                       cu_v[0, pl.ds(j * 16, 16)])
                plsc.addupdate_scatter(pacc_v, [ir + N],
                                       cu_v[1, pl.ds(j * 16, 16)])
                plsc.addupdate_scatter(pacc_v, [ir + 2 * N],
                                       cu_v[2, pl.ds(j * 16, 16)])

    plsc.subcore_barrier()

    @pl.loop(0, _ZIT)
    def _(i):
        j = s + NS * i

        @pl.when(j < _ZCH)
        def _():
            pltpu.sync_copy(acc.at[pl.ds(j * _ZR, _ZR)],
                            agg_hbm.at[pl.ds(c * NP + j * _ZR, _ZR)])

    w = s * NC + c
    pltpu.sync_copy(pacc_v, pout_hbm.at[w])


def _sc_scatter(m, cu, ridx2, zrows):
    mesh = plsc.VectorSubcoreMesh(core_axis_name="c", subcore_axis_name="s")
    f = pl.kernel(
        _scatter_body,
        out_type=(
            jax.ShapeDtypeStruct((2 * NP, D), jnp.float32),
            jax.ShapeDtypeStruct((NW, 3 * N), jnp.float32),
        ),
        mesh=mesh,
        compiler_params=pltpu.CompilerParams(needs_layout_passes=False),
        scratch_types=[
            pltpu.VMEM((CH,), jnp.int32),
            pltpu.VMEM((CH, D), jnp.float32),
            pltpu.VMEM((4, CH), jnp.float32),
            pltpu.VMEM((3 * N,), jnp.float32),
            pltpu.SemaphoreType.DMA,
            pltpu.VMEM_SHARED((N, D), jnp.float32),
        ],
    )
    return f(m, cu, ridx2, zrows)


def _silu(x):
    return x * jax.nn.sigmoid(x)


def _edge_body(gr, gc, rad, ea, w1a, w1b, w1n, w1e, b1, w2, b2, w5, b5, w6t,
               b6, m_out, cu_out):
    radt = jnp.transpose(rad[...], (1, 0))
    rn = radt[:, 0:1]
    rnorm = radt[:, 1:4]
    eat = jnp.transpose(ea[...], (1, 0))

    x = (jnp.dot(gr[...], w1a[...], preferred_element_type=jnp.float32)
         + jnp.dot(gc[...], w1b[...], preferred_element_type=jnp.float32)
         + rn * w1n[...]
         + jnp.dot(eat, w1e[...], preferred_element_type=jnp.float32)
         + b1[...])
    x = _silu(x)
    m = _silu(jnp.dot(x, w2[...], preferred_element_type=jnp.float32) + b2[...])
    y = _silu(jnp.dot(m, w5[...], preferred_element_type=jnp.float32) + b5[...])
    cd = jnp.sum(y * w6t[...], axis=1, keepdims=True) + b6[...]
    m_out[...] = m
    cu4 = jnp.concatenate(
        [cd * rnorm, jnp.zeros((TE, 1), jnp.float32)], axis=1)
    cu_out[...] = jnp.transpose(cu4, (1, 0))


def _tc_edge(g, rad, ea, w1a, w1b, w1n, w1e, b1, w2, b2, w5, b5, w6t, b6):
    grid = (SE // TE,)
    full = lambda shape: pl.BlockSpec(shape, lambda i: tuple(0 for _ in shape))
    return pl.pallas_call(
        _edge_body,
        grid=grid,
        in_specs=[
            pl.BlockSpec((TE, D), lambda i: (i, 0)),
            pl.BlockSpec((TE, D), lambda i: (i + SE // TE, 0)),
            pl.BlockSpec((4, TE), lambda i: (0, i)),
            pl.BlockSpec((ED, TE), lambda i: (0, i)),
            full((D, D)), full((D, D)), full((1, D)), full((ED, D)),
            full((1, D)), full((D, D)), full((1, D)), full((D, D)),
            full((1, D)), full((1, D)), full((1, 1)),
        ],
        out_specs=[
            pl.BlockSpec((TE, D), lambda i: (i, 0)),
            pl.BlockSpec((4, TE), lambda i: (0, i)),
        ],
        out_shape=[
            jax.ShapeDtypeStruct((SE, D), jnp.float32),
            jax.ShapeDtypeStruct((4, SE), jnp.float32),
        ],
        compiler_params=pltpu.CompilerParams(
            dimension_semantics=("arbitrary",)),
    )(g, g, rad, ea, w1a, w1b, w1n, w1e, b1, w2, b2, w5, b5, w6t, b6)


def _node_body(h, *rest):
    parts = rest[:2 * S]
    w3a, w3b, b3, w4, b4 = rest[2 * S:2 * S + 5]
    hn = rest[2 * S + 5]
    agg = parts[0][...]
    for pr in parts[1:]:
        agg = agg + pr[...]
    x = _silu(jnp.dot(h[...], w3a[...], preferred_element_type=jnp.float32)
              + jnp.dot(agg, w3b[...], preferred_element_type=jnp.float32)
              + b3[...])
    hn[...] = h[...] + jnp.dot(x, w4[...],
                               preferred_element_type=jnp.float32) + b4[...]


def _tc_node(h, parts_list, w3a, w3b, b3, w4, b4):
    grid = (NP // TN,)
    full = lambda shape: pl.BlockSpec(shape, lambda i: tuple(0 for _ in shape))
    part_specs = []
    for _ in parts_list:
        part_specs.append(pl.BlockSpec((TN, D), lambda i: (i, 0)))
        part_specs.append(pl.BlockSpec((TN, D), lambda i: (i + NP // TN, 0)))
    part_args = []
    for pr in parts_list:
        part_args.extend([pr, pr])
    return pl.pallas_call(
        _node_body,
        grid=grid,
        in_specs=[pl.BlockSpec((TN, D), lambda i: (i, 0))] + part_specs + [
            full((D, D)), full((D, D)), full((1, D)), full((D, D)),
            full((1, D)),
        ],
        out_specs=pl.BlockSpec((TN, D), lambda i: (i, 0)),
        out_shape=jax.ShapeDtypeStruct((NP, D), jnp.float32),
        compiler_params=pltpu.CompilerParams(
            dimension_semantics=("arbitrary",)),
    )(h, *part_args, w3a, w3b, b3, w4, b4)


def kernel(h, pos, edge_attr, W1, b1, W2, b2, W3, b3, W4, b4, W5, b5, W6, b6,
           edge_index):
    post = pos.T.reshape(3 * N)
    idxr2 = edge_index[0].reshape(S, _CHUNKS, CH)
    idxc2 = edge_index[1].reshape(S, _CHUNKS, CH)

    w1a = W1[0:D]
    w1b = W1[D:2 * D]
    w1n = W1[2 * D:2 * D + 1]
    w1e = W1[2 * D + 1:]
    eat = edge_attr.T.reshape(ED, S, SE)
    zrows = jnp.zeros((_ZR, D), jnp.float32)

    parts_list = []
    pparts_list = []
    for si in range(S):
        g, rad = _sc_gather(h, post, idxr2[si], idxc2[si])
        m, cu = _tc_edge(g, rad, eat[:, si], w1a, w1b, w1n, w1e,
                         b1.reshape(1, D), W2, b2.reshape(1, D),
                         W5, b5.reshape(1, D),
                         W6.reshape(1, D), b6.reshape(1, 1))
        parts, pparts = _sc_scatter(m, cu, idxr2[si], zrows)
        parts_list.append(parts)
        pparts_list.append(pparts)

    hp = jnp.concatenate([h, jnp.zeros((NP - N, D), jnp.float32)], axis=0)
    h_new = _tc_node(hp, parts_list, W3[0:D], W3[D:2 * D],
                     b3.reshape(1, D), W4, b4.reshape(1, D))
    pd = pparts_list[0].sum(axis=0)
    for pp in pparts_list[1:]:
        pd = pd + pp.sum(axis=0)
    pos_new = pos + pd.reshape(3, N).T
    return (h_new[:N], pos_new)


# speedup vs baseline: 1.5628x; 1.0018x over previous
import jax
import jax.numpy as jnp
from jax import lax
from jax.experimental import pallas as pl
from jax.experimental.pallas import tpu as pltpu
from jax.experimental.pallas import tpu_sc as plsc

N = 10000
E = 320000
D = 128
ED = 4
CH = 128
NC = 2
NS = 16
NW = NC * NS

TE = 3200
NP = 10240
TN = 2048

S = 5
SE = E // S
_CHUNKS = SE // CH
_ITERS = -(-_CHUNKS // NW)
_ZR = 200
_ZCH = N // _ZR
_ZIT = -(-_ZCH // NS)


def _rsqrt(x):
    i = plsc.bitcast(x, jnp.int32)
    i = jnp.full((16,), 0x5F3759DF, jnp.int32) - (i >> 1)
    y = plsc.bitcast(i, jnp.float32)
    half = x * 0.5
    for _ in range(3):
        y = y * (1.5 - half * y * y)
    return y


def _gather_body(htbl, post, idxr2, idxc2, g_hbm, rad_hbm,
                 pos_v, idxr_v, idxc_v, rowsr_v, rowsc_v, rad_v, semr, semc):
    c = lax.axis_index("c")
    s = lax.axis_index("s")

    pltpu.sync_copy(post, pos_v)

    @pl.loop(0, _ITERS)
    def _(t):
        k = ((t * NS + s) * NC + c).astype(jnp.int32)

        @pl.when(k < _CHUNKS)
        def _():
            pltpu.sync_copy(idxr2.at[k], idxr_v)
            pltpu.sync_copy(idxc2.at[k], idxc_v)
            cpr = pltpu.async_copy(htbl.at[idxr_v], rowsr_v, semr)
            cpc = pltpu.async_copy(htbl.at[idxc_v], rowsc_v, semc)
            for j in range(CH // 16):
                ir = idxr_v[pl.ds(j * 16, 16)]
                ic = idxc_v[pl.ds(j * 16, 16)]
                rx = (plsc.load_gather(pos_v, [ir])
                      - plsc.load_gather(pos_v, [ic]))
                ry = (plsc.load_gather(pos_v, [ir + N])
                      - plsc.load_gather(pos_v, [ic + N]))
                rz = (plsc.load_gather(pos_v, [ir + 2 * N])
                      - plsc.load_gather(pos_v, [ic + 2 * N]))
                rn2 = rx * rx + ry * ry + rz * rz
                inv = _rsqrt(rn2)
                rn = jnp.maximum(rn2 * inv, 1e-8)
                rad_v[0, pl.ds(j * 16, 16)] = rn
                rad_v[1, pl.ds(j * 16, 16)] = rx * inv
                rad_v[2, pl.ds(j * 16, 16)] = ry * inv
                rad_v[3, pl.ds(j * 16, 16)] = rz * inv
            cpr.wait()
            cpc.wait()
            pltpu.sync_copy(rowsr_v, g_hbm.at[pl.ds(k * CH, CH)])
            pltpu.sync_copy(rowsc_v, g_hbm.at[pl.ds(SE + k * CH, CH)])
            pltpu.sync_copy(rad_v, rad_hbm.at[:, pl.ds(k * CH, CH)])


def _sc_gather(htbl, post, idxr2, idxc2):
    mesh = plsc.VectorSubcoreMesh(core_axis_name="c", subcore_axis_name="s")
    f = pl.kernel(
        _gather_body,
        out_type=(
            jax.ShapeDtypeStruct((2 * SE, D), jnp.float32),
            jax.ShapeDtypeStruct((4, SE), jnp.float32),
        ),
        mesh=mesh,
        compiler_params=pltpu.CompilerParams(needs_layout_passes=False),
        scratch_types=[
            pltpu.VMEM((3 * N,), jnp.float32),
            pltpu.VMEM((CH,), jnp.int32),
            pltpu.VMEM((CH,), jnp.int32),
            pltpu.VMEM((CH, D), jnp.float32),
            pltpu.VMEM((CH, D), jnp.float32),
            pltpu.VMEM((4, CH), jnp.float32),
            pltpu.SemaphoreType.DMA,
            pltpu.SemaphoreType.DMA,
        ],
    )
    return f(htbl, post, idxr2, idxc2)


def _scatter_body(m_hbm, cu_hbm, ridx2, z_hbm, agg_hbm, pout_hbm,
                  idx_v, rows_v, cu_v, pacc_v, sem, acc):
    c = lax.axis_index("c")
    s = lax.axis_index("s")

    @pl.loop(0, _ZIT)
    def _(i):
        j = s + NS * i

        @pl.when(j < _ZCH)
        def _():
            pltpu.sync_copy(z_hbm, acc.at[pl.ds(j * _ZR, _ZR)])

    @pl.loop(0, 3 * N // 16)
    def _(i):
        pacc_v[pl.ds(i * 16, 16)] = jnp.zeros((16,), jnp.float32)

    plsc.subcore_barrier()

    @pl.loop(0, _ITERS)
    def _(t):
        k = ((t * NS + s) * NC + c).astype(jnp.int32)

        @pl.when(k < _CHUNKS)
        def _():
            pltpu.sync_copy(ridx2.at[k], idx_v)
            pltpu.sync_copy(m_hbm.at[pl.ds(k * CH, CH)], rows_v)
            pltpu.sync_copy(cu_hbm.at[:, pl.ds(k * CH, CH)], cu_v)
            pltpu.sync_copy(rows_v, acc.at[idx_v], add=True)
            for j in range(CH // 16):
                ir = idx_v[pl.ds(j * 16, 16)]
                plsc.addupdate_scatter(pacc_v, [ir],
                                       cu_v[0, pl.ds(j * 16, 16)])
                plsc.addupdate_scatter(pacc_v, [ir + N],
                                       cu_v[1, pl.ds(j * 16, 16)])
                plsc.addupdate_scatter(pacc_v, [ir + 2 * N],
                                       cu_v[2, pl.ds(j * 16, 16)])

    plsc.subcore_barrier()

    @pl.loop(0, _ZIT)
    def _(i):
        j = s + NS * i

        @pl.when(j < _ZCH)
        def _():
            pltpu.sync_copy(acc.at[pl.ds(j * _ZR, _ZR)],
                            agg_hbm.at[pl.ds(c * NP + j * _ZR, _ZR)])

    w = s * NC + c
    pltpu.sync_copy(pacc_v, pout_hbm.at[w])


def _sc_scatter(m, cu, ridx2, zrows):
    mesh = plsc.VectorSubcoreMesh(core_axis_name="c", subcore_axis_name="s")
    f = pl.kernel(
        _scatter_body,
        out_type=(
            jax.ShapeDtypeStruct((2 * NP, D), jnp.float32),
            jax.ShapeDtypeStruct((NW, 3 * N), jnp.float32),
        ),
        mesh=mesh,
        compiler_params=pltpu.CompilerParams(needs_layout_passes=False),
        scratch_types=[
            pltpu.VMEM((CH,), jnp.int32),
            pltpu.VMEM((CH, D), jnp.float32),
            pltpu.VMEM((4, CH), jnp.float32),
            pltpu.VMEM((3 * N,), jnp.float32),
            pltpu.SemaphoreType.DMA,
            pltpu.VMEM_SHARED((N, D), jnp.float32),
        ],
    )
    return f(m, cu, ridx2, zrows)


def _silu(x):
    return x * jax.nn.sigmoid(x)


def _edge_body(gr, gc, rad, ea, w1a, w1b, w1n, w1e, b1, w2, b2, w5, b5, w6t,
               b6, m_out, cu_out):
    radt = jnp.transpose(rad[...], (1, 0))
    rn = radt[:, 0:1]
    rnorm = radt[:, 1:4]
    eat = jnp.transpose(ea[...], (1, 0))

    x = (jnp.dot(gr[...], w1a[...], preferred_element_type=jnp.float32)
         + jnp.dot(gc[...], w1b[...], preferred_element_type=jnp.float32)
         + rn * w1n[...]
         + jnp.dot(eat, w1e[...], preferred_element_type=jnp.float32)
         + b1[...])
    x = _silu(x)
    m = _silu(jnp.dot(x, w2[...], preferred_element_type=jnp.float32) + b2[...])
    y = _silu(jnp.dot(m, w5[...], preferred_element_type=jnp.float32) + b5[...])
    cd = jnp.sum(y * w6t[...], axis=1, keepdims=True) + b6[...]
    m_out[...] = m
    cu4 = jnp.concatenate(
        [cd * rnorm, jnp.zeros((TE, 1), jnp.float32)], axis=1)
    cu_out[...] = jnp.transpose(cu4, (1, 0))


def _tc_edge(g, rad, ea, w1a, w1b, w1n, w1e, b1, w2, b2, w5, b5, w6t, b6):
    grid = (SE // TE,)
    full = lambda shape: pl.BlockSpec(shape, lambda i: tuple(0 for _ in shape))
    return pl.pallas_call(
        _edge_body,
        grid=grid,
        in_specs=[
            pl.BlockSpec((TE, D), lambda i: (i, 0)),
            pl.BlockSpec((TE, D), lambda i: (i + SE // TE, 0)),
            pl.BlockSpec((4, TE), lambda i: (0, i)),
            pl.BlockSpec((ED, TE), lambda i: (0, i)),
            full((D, D)), full((D, D)), full((1, D)), full((ED, D)),
            full((1, D)), full((D, D)), full((1, D)), full((D, D)),
            full((1, D)), full((1, D)), full((1, 1)),
        ],
        out_specs=[
            pl.BlockSpec((TE, D), lambda i: (i, 0)),
            pl.BlockSpec((4, TE), lambda i: (0, i)),
        ],
        out_shape=[
            jax.ShapeDtypeStruct((SE, D), jnp.float32),
            jax.ShapeDtypeStruct((4, SE), jnp.float32),
        ],
        compiler_params=pltpu.CompilerParams(
            dimension_semantics=("arbitrary",)),
    )(g, g, rad, ea, w1a, w1b, w1n, w1e, b1, w2, b2, w5, b5, w6t, b6)


def _node_body(h, *rest):
    parts = rest[:2 * S]
    w3a, w3b, b3, w4, b4 = rest[2 * S:2 * S + 5]
    hn = rest[2 * S + 5]
    agg = parts[0][...]
    for pr in parts[1:]:
        agg = agg + pr[...]
    x = _silu(jnp.dot(h[...], w3a[...], preferred_element_type=jnp.float32)
              + jnp.dot(agg, w3b[...], preferred_element_type=jnp.float32)
              + b3[...])
    hn[...] = h[...] + jnp.dot(x, w4[...],
                               preferred_element_type=jnp.float32) + b4[...]


def _tc_node(h, parts_list, w3a, w3b, b3, w4, b4):
    grid = (NP // TN,)
    full = lambda shape: pl.BlockSpec(shape, lambda i: tuple(0 for _ in shape))
    part_specs = []
    for _ in parts_list:
        part_specs.append(pl.BlockSpec((TN, D), lambda i: (i, 0)))
        part_specs.append(pl.BlockSpec((TN, D), lambda i: (i + NP // TN, 0)))
    part_args = []
    for pr in parts_list:
        part_args.extend([pr, pr])
    return pl.pallas_call(
        _node_body,
        grid=grid,
        in_specs=[pl.BlockSpec((TN, D), lambda i: (i, 0))] + part_specs + [
            full((D, D)), full((D, D)), full((1, D)), full((D, D)),
            full((1, D)),
        ],
        out_specs=pl.BlockSpec((TN, D), lambda i: (i, 0)),
        out_shape=jax.ShapeDtypeStruct((NP, D), jnp.float32),
        compiler_params=pltpu.CompilerParams(
            dimension_semantics=("arbitrary",)),
    )(h, *part_args, w3a, w3b, b3, w4, b4)


def kernel(h, pos, edge_attr, W1, b1, W2, b2, W3, b3, W4, b4, W5, b5, W6, b6,
           edge_index):
    post = pos.T.reshape(3 * N)
    idxr2 = edge_index[0].reshape(S, _CHUNKS, CH)
    idxc2 = edge_index[1].reshape(S, _CHUNKS, CH)

    w1a = W1[0:D]
    w1b = W1[D:2 * D]
    w1n = W1[2 * D:2 * D + 1]
    w1e = W1[2 * D + 1:]
    eat = edge_attr.T.reshape(ED, S, SE)
    zrows = jnp.zeros((_ZR, D), jnp.float32)

    parts_list = []
    pparts_list = []
    for si in range(S):
        g, rad = _sc_gather(h, post, idxr2[si], idxc2[si])
        m, cu = _tc_edge(g, rad, eat[:, si], w1a, w1b, w1n, w1e,
                         b1.reshape(1, D), W2, b2.reshape(1, D),
                         W5, b5.reshape(1, D),
                         W6.reshape(1, D), b6.reshape(1, 1))
        parts, pparts = _sc_scatter(m, cu, idxr2[si], zrows)
        parts_list.append(parts)
        pparts_list.append(pparts)

    hp = jnp.concatenate([h, jnp.zeros((NP - N, D), jnp.float32)], axis=0)
    h_new = _tc_node(hp, parts_list, W3[0:D], W3[D:2 * D],
                     b3.reshape(1, D), W4, b4.reshape(1, D))
    pd = pparts_list[0].sum(axis=0)
    for pp in pparts_list[1:]:
        pd = pd + pp.sum(axis=0)
    pos_new = pos + pd.reshape(3, N).T
    return (h_new[:N], pos_new)


# speedup vs baseline: 1.5837x; 1.0134x over previous
import jax
import jax.numpy as jnp
from jax import lax
from jax.experimental import pallas as pl
from jax.experimental.pallas import tpu as pltpu
from jax.experimental.pallas import tpu_sc as plsc

N = 10000
E = 320000
D = 128
ED = 4
CH = 128
NC = 2
NS = 16
NW = NC * NS

TE = 3200
NP = 10240
TN = 2048

S = 5
SE = E // S
_CHUNKS = SE // CH
_ITERS = -(-_CHUNKS // NW)
_ZR = 200
_ZCH = N // _ZR
_ZIT = -(-_ZCH // NS)


def _rsqrt(x):
    i = plsc.bitcast(x, jnp.int32)
    i = jnp.full((16,), 0x5F3759DF, jnp.int32) - (i >> 1)
    y = plsc.bitcast(i, jnp.float32)
    half = x * 0.5
    for _ in range(3):
        y = y * (1.5 - half * y * y)
    return y


def _gather_body(htbl, post, idxr2, idxc2, g_hbm, rad_hbm,
                 pos_v, idxr_v, idxc_v, rowsr_v, rowsc_v, rad_v, semr, semc):
    c = lax.axis_index("c")
    s = lax.axis_index("s")

    pltpu.sync_copy(post, pos_v)

    @pl.loop(0, _ITERS)
    def _(t):
        k = ((t * NS + s) * NC + c).astype(jnp.int32)

        @pl.when(k < _CHUNKS)
        def _():
            pltpu.sync_copy(idxr2.at[k], idxr_v)
            pltpu.sync_copy(idxc2.at[k], idxc_v)
            cpr = pltpu.async_copy(htbl.at[idxr_v], rowsr_v, semr)
            cpc = pltpu.async_copy(htbl.at[idxc_v], rowsc_v, semc)
            for j in range(CH // 16):
                ir = idxr_v[pl.ds(j * 16, 16)]
                ic = idxc_v[pl.ds(j * 16, 16)]
                rx = (plsc.load_gather(pos_v, [ir])
                      - plsc.load_gather(pos_v, [ic]))
                ry = (plsc.load_gather(pos_v, [ir + N])
                      - plsc.load_gather(pos_v, [ic + N]))
                rz = (plsc.load_gather(pos_v, [ir + 2 * N])
                      - plsc.load_gather(pos_v, [ic + 2 * N]))
                rn2 = rx * rx + ry * ry + rz * rz
                inv = _rsqrt(rn2)
                rn = jnp.maximum(rn2 * inv, 1e-8)
                rad_v[0, pl.ds(j * 16, 16)] = rn
                rad_v[1, pl.ds(j * 16, 16)] = rx * inv
                rad_v[2, pl.ds(j * 16, 16)] = ry * inv
                rad_v[3, pl.ds(j * 16, 16)] = rz * inv
            cpr.wait()
            cpc.wait()
            pltpu.sync_copy(rowsr_v, g_hbm.at[pl.ds(k * CH, CH)])
            pltpu.sync_copy(rowsc_v, g_hbm.at[pl.ds(SE + k * CH, CH)])
            pltpu.sync_copy(rad_v, rad_hbm.at[:, pl.ds(k * CH, CH)])


def _sc_gather(htbl, post, idxr2, idxc2):
    mesh = plsc.VectorSubcoreMesh(core_axis_name="c", subcore_axis_name="s")
    f = pl.kernel(
        _gather_body,
        out_type=(
            jax.ShapeDtypeStruct((2 * SE, D), jnp.float32),
            jax.ShapeDtypeStruct((4, SE), jnp.float32),
        ),
        mesh=mesh,
        compiler_params=pltpu.CompilerParams(needs_layout_passes=False),
        scratch_types=[
            pltpu.VMEM((3 * N,), jnp.float32),
            pltpu.VMEM((CH,), jnp.int32),
            pltpu.VMEM((CH,), jnp.int32),
            pltpu.VMEM((CH, D), jnp.float32),
            pltpu.VMEM((CH, D), jnp.float32),
            pltpu.VMEM((4, CH), jnp.float32),
            pltpu.SemaphoreType.DMA,
            pltpu.SemaphoreType.DMA,
        ],
    )
    return f(htbl, post, idxr2, idxc2)


def _scatter_body(m_hbm, cui_hbm, z_hbm, agg_hbm, pout_hbm,
                  rows_v, cui_v, pacc_v, sem, acc):
    c = lax.axis_index("c")
    s = lax.axis_index("s")

    @pl.loop(0, _ZIT)
    def _(i):
        j = s + NS * i

        @pl.when(j < _ZCH)
        def _():
            pltpu.sync_copy(z_hbm, acc.at[pl.ds(j * _ZR, _ZR)])

    @pl.loop(0, 3 * N // 16)
    def _(i):
        pacc_v[pl.ds(i * 16, 16)] = jnp.zeros((16,), jnp.float32)

    plsc.subcore_barrier()

    @pl.loop(0, _ITERS)
    def _(t):
        k = ((t * NS + s) * NC + c).astype(jnp.int32)

        @pl.when(k < _CHUNKS)
        def _():
            pltpu.sync_copy(cui_hbm.at[:, pl.ds(k * CH, CH)], cui_v)
            pltpu.sync_copy(m_hbm.at[pl.ds(k * CH, CH)], rows_v)
            pltpu.sync_copy(rows_v, acc.at[cui_v.at[4]], add=True)
            for j in range(CH // 16):
                ir = cui_v[4, pl.ds(j * 16, 16)]
                plsc.addupdate_scatter(
                    pacc_v, [ir],
                    plsc.bitcast(cui_v[0, pl.ds(j * 16, 16)], jnp.float32))
                plsc.addupdate_scatter(
                    pacc_v, [ir + N],
                    plsc.bitcast(cui_v[1, pl.ds(j * 16, 16)], jnp.float32))
                plsc.addupdate_scatter(
                    pacc_v, [ir + 2 * N],
                    plsc.bitcast(cui_v[2, pl.ds(j * 16, 16)], jnp.float32))

    plsc.subcore_barrier()

    @pl.loop(0, _ZIT)
    def _(i):
        j = s + NS * i

        @pl.when(j < _ZCH)
        def _():
            pltpu.sync_copy(acc.at[pl.ds(j * _ZR, _ZR)],
                            agg_hbm.at[pl.ds(c * NP + j * _ZR, _ZR)])

    w = s * NC + c
    pltpu.sync_copy(pacc_v, pout_hbm.at[w])


def _sc_scatter(m, cui, zrows):
    mesh = plsc.VectorSubcoreMesh(core_axis_name="c", subcore_axis_name="s")
    f = pl.kernel(
        _scatter_body,
        out_type=(
            jax.ShapeDtypeStruct((2 * NP, D), jnp.float32),
            jax.ShapeDtypeStruct((NW, 3 * N), jnp.float32),
        ),
        mesh=mesh,
        compiler_params=pltpu.CompilerParams(needs_layout_passes=False),
        scratch_types=[
            pltpu.VMEM((CH, D), jnp.float32),
            pltpu.VMEM((5, CH), jnp.int32),
            pltpu.VMEM((3 * N,), jnp.float32),
            pltpu.SemaphoreType.DMA,
            pltpu.VMEM_SHARED((N, D), jnp.float32),
        ],
    )
    return f(m, cui, zrows)


def _silu(x):
    return x * jax.nn.sigmoid(x)


def _edge_body(gr, gc, rad, ea, w1a, w1b, w1n, w1e, b1, w2, b2, w5, b5, w6t,
               b6, m_out, cu_out):
    radt = jnp.transpose(rad[...], (1, 0))
    rn = radt[:, 0:1]
    rnorm = radt[:, 1:4]
    eat = jnp.transpose(ea[...], (1, 0))

    x = (jnp.dot(gr[...], w1a[...], preferred_element_type=jnp.float32)
         + jnp.dot(gc[...], w1b[...], preferred_element_type=jnp.float32)
         + rn * w1n[...]
         + jnp.dot(eat, w1e[...], preferred_element_type=jnp.float32)
         + b1[...])
    x = _silu(x)
    m = _silu(jnp.dot(x, w2[...], preferred_element_type=jnp.float32) + b2[...])
    y = _silu(jnp.dot(m, w5[...], preferred_element_type=jnp.float32) + b5[...])
    cd = jnp.sum(y * w6t[...], axis=1, keepdims=True) + b6[...]
    m_out[...] = m
    cu4 = jnp.concatenate(
        [cd * rnorm, jnp.zeros((TE, 1), jnp.float32)], axis=1)
    cu_out[...] = jnp.transpose(cu4, (1, 0))


def _tc_edge(g, rad, ea, w1a, w1b, w1n, w1e, b1, w2, b2, w5, b5, w6t, b6):
    grid = (SE // TE,)
    full = lambda shape: pl.BlockSpec(shape, lambda i: tuple(0 for _ in shape))
    return pl.pallas_call(
        _edge_body,
        grid=grid,
        in_specs=[
            pl.BlockSpec((TE, D), lambda i: (i, 0)),
            pl.BlockSpec((TE, D), lambda i: (i + SE // TE, 0)),
            pl.BlockSpec((4, TE), lambda i: (0, i)),
            pl.BlockSpec((ED, TE), lambda i: (0, i)),
            full((D, D)), full((D, D)), full((1, D)), full((ED, D)),
            full((1, D)), full((D, D)), full((1, D)), full((D, D)),
            full((1, D)), full((1, D)), full((1, 1)),
        ],
        out_specs=[
            pl.BlockSpec((TE, D), lambda i: (i, 0)),
            pl.BlockSpec((4, TE), lambda i: (0, i)),
        ],
        out_shape=[
            jax.ShapeDtypeStruct((SE, D), jnp.float32),
            jax.ShapeDtypeStruct((4, SE), jnp.float32),
        ],
        compiler_params=pltpu.CompilerParams(
            dimension_semantics=("arbitrary",)),
    )(g, g, rad, ea, w1a, w1b, w1n, w1e, b1, w2, b2, w5, b5, w6t, b6)


def _node_body(h, *rest):
    parts = rest[:2 * S]
    w3a, w3b, b3, w4, b4 = rest[2 * S:2 * S + 5]
    hn = rest[2 * S + 5]
    agg = parts[0][...]
    for pr in parts[1:]:
        agg = agg + pr[...]
    x = _silu(jnp.dot(h[...], w3a[...], preferred_element_type=jnp.float32)
              + jnp.dot(agg, w3b[...], preferred_element_type=jnp.float32)
              + b3[...])
    hn[...] = h[...] + jnp.dot(x, w4[...],
                               preferred_element_type=jnp.float32) + b4[...]


def _tc_node(h, parts_list, w3a, w3b, b3, w4, b4):
    grid = (NP // TN,)
    full = lambda shape: pl.BlockSpec(shape, lambda i: tuple(0 for _ in shape))
    part_specs = []
    for _ in parts_list:
        part_specs.append(pl.BlockSpec((TN, D), lambda i: (i, 0)))
        part_specs.append(pl.BlockSpec((TN, D), lambda i: (i + NP // TN, 0)))
    part_args = []
    for pr in parts_list:
        part_args.extend([pr, pr])
    return pl.pallas_call(
        _node_body,
        grid=grid,
        in_specs=[pl.BlockSpec((TN, D), lambda i: (i, 0))] + part_specs + [
            full((D, D)), full((D, D)), full((1, D)), full((D, D)),
            full((1, D)),
        ],
        out_specs=pl.BlockSpec((TN, D), lambda i: (i, 0)),
        out_shape=jax.ShapeDtypeStruct((NP, D), jnp.float32),
        compiler_params=pltpu.CompilerParams(
            dimension_semantics=("arbitrary",)),
    )(h, *part_args, w3a, w3b, b3, w4, b4)


def kernel(h, pos, edge_attr, W1, b1, W2, b2, W3, b3, W4, b4, W5, b5, W6, b6,
           edge_index):
    post = pos.T.reshape(3 * N)
    idxr2 = edge_index[0].reshape(S, _CHUNKS, CH)
    idxc2 = edge_index[1].reshape(S, _CHUNKS, CH)

    w1a = W1[0:D]
    w1b = W1[D:2 * D]
    w1n = W1[2 * D:2 * D + 1]
    w1e = W1[2 * D + 1:]
    eat = edge_attr.T.reshape(ED, S, SE)
    zrows = jnp.zeros((_ZR, D), jnp.float32)

    parts_list = []
    pparts_list = []
    for si in range(S):
        g, rad = _sc_gather(h, post, idxr2[si], idxc2[si])
        m, cu = _tc_edge(g, rad, eat[:, si], w1a, w1b, w1n, w1e,
                         b1.reshape(1, D), W2, b2.reshape(1, D),
                         W5, b5.reshape(1, D),
                         W6.reshape(1, D), b6.reshape(1, 1))
        cui = jnp.concatenate(
            [lax.bitcast_convert_type(cu, jnp.int32),
             idxr2[si].reshape(1, SE)], axis=0)
        parts, pparts = _sc_scatter(m, cui, zrows)
        parts_list.append(parts)
        pparts_list.append(pparts)

    hp = jnp.concatenate([h, jnp.zeros((NP - N, D), jnp.float32)], axis=0)
    h_new = _tc_node(hp, parts_list, W3[0:D], W3[D:2 * D],
                     b3.reshape(1, D), W4, b4.reshape(1, D))
    pd = pparts_list[0].sum(axis=0)
    for pp in pparts_list[1:]:
        pd = pd + pp.sum(axis=0)
    pos_new = pos + pd.reshape(3, N).T
    return (h_new[:N], pos_new)
